# baseline (speedup 1.0000x reference)
import functools

import jax
import jax.numpy as jnp
from jax import lax
from jax.experimental import pallas as pl
from jax.experimental.pallas import tpu as pltpu
from jax.experimental.pallas import tpu_sc as plsc

_NC = 2
_NS = 16
_NW = _NC * _NS
_L = 16
_BINS = 10
_ROWS = 2 * _BINS
_CHUNK = 8192

_LOG1P = (
    5.621959008883515e-07, 0.9999574870750662, -0.4992065685478449,
    0.32697310001386687, -0.2228362583280196, 0.13076503250423846,
    -0.052624851367851076, 0.010119082927824848,
)


def _log1p_poly(u):
    acc = jnp.float32(_LOG1P[7])
    for k in range(6, -1, -1):
        acc = acc * u + jnp.float32(_LOG1P[k])
    return acc


def _hist_body(n, x_hbm, t_hbm, w_hbm, parts_hbm, xb, tb, wb, hist, s0, s1):
    wid = lax.axis_index("s") * _NC + lax.axis_index("c")
    per_w = n // _NW
    nchunks = per_w // _CHUNK
    base = wid * per_w
    sems = (s0, s1)
    lane = lax.iota(jnp.int32, _L)

    for r in range(_ROWS):
        hist[pl.ds(r * _L, _L)] = jnp.zeros((_L,), jnp.float32)

    def start(c, buf):
        off = base + c * _CHUNK
        sem = sems[buf]
        pltpu.make_async_copy(x_hbm.at[pl.ds(off, _CHUNK)], xb.at[buf], sem).start()
        pltpu.make_async_copy(t_hbm.at[pl.ds(off, _CHUNK)], tb.at[buf], sem).start()
        pltpu.make_async_copy(w_hbm.at[pl.ds(off, _CHUNK)], wb.at[buf], sem).start()

    def wait(buf):
        sem = sems[buf]
        pltpu.make_async_copy(x_hbm.at[pl.ds(0, _CHUNK)], xb.at[buf], sem).wait()
        pltpu.make_async_copy(t_hbm.at[pl.ds(0, _CHUNK)], tb.at[buf], sem).wait()
        pltpu.make_async_copy(w_hbm.at[pl.ds(0, _CHUNK)], wb.at[buf], sem).wait()

    def process(buf):
        def vbody(v, carry):
            s = v * _L
            x = xb[buf, pl.ds(s, _L)]
            tf = tb[buf, pl.ds(s, _L)].astype(jnp.float32)
            lw = wb[buf, pl.ds(s, _L)]
            ax = jnp.abs(x)
            e = jnp.exp(-ax)
            r = 1.0 / (1.0 + e)
            p = jnp.where(x >= 0, r, e * r)
            g = jnp.abs(p - tf)
            bi = jnp.maximum(jnp.minimum((g * 10.0).astype(jnp.int32), _BINS - 1), 0)
            pe = jnp.maximum(x, 0.0) - x * tf + _log1p_poly(e)
            valid = lw > 0
            cnt_inc = jnp.where(valid, 1.0, 0.0).astype(jnp.float32)
            pe_m = jnp.where(valid, pe, 0.0).astype(jnp.float32)
            flat = bi * _L + lane
            plsc.addupdate_scatter(hist, [flat], cnt_inc)
            plsc.addupdate_scatter(hist, [flat + _BINS * _L], pe_m)
            return carry

        lax.fori_loop(0, _CHUNK // _L, vbody, 0)

    start(0, 0)

    def cbody(i, carry):
        c0 = 2 * i
        wait(0)
        start(c0 + 1, 1)
        process(0)
        wait(1)

        @pl.when(c0 + 2 < nchunks)
        def _():
            start(c0 + 2, 0)

        process(1)
        return carry

    lax.fori_loop(0, nchunks // 2, cbody, 0)
    pltpu.sync_copy(hist, parts_hbm.at[wid])


def _combine_body(parts_hbm, out_hbm, pv, ov, sem):
    wid = lax.axis_index("s") * _NC + lax.axis_index("c")

    @pl.when(wid == 0)
    def _():
        pltpu.make_async_copy(parts_hbm, pv, sem).start()
        pltpu.make_async_copy(parts_hbm, pv, sem).wait()

        def wbody(w, accs):
            return tuple(
                accs[r] + pv[w, pl.ds(r * _L, _L)] for r in range(_ROWS)
            )

        zero = jnp.zeros((_L,), jnp.float32)
        accs = lax.fori_loop(0, _NW, wbody, tuple(zero for _ in range(_ROWS)))

        nbv = jnp.zeros((_L,), jnp.float32)
        contribv = jnp.zeros((_L,), jnp.float32)
        for b in range(_BINS):
            cntv = jnp.broadcast_to(jnp.sum(accs[b]), (_L,))
            bsv = jnp.broadcast_to(jnp.sum(accs[b + _BINS]), (_L,))
            has = cntv > 0
            nbv = nbv + jnp.where(has, 1.0, 0.0).astype(jnp.float32)
            contribv = contribv + jnp.where(
                has, bsv / jnp.maximum(cntv, 1.0), 0.0
            ).astype(jnp.float32)
        lossv = jnp.where(nbv > 0, contribv / jnp.maximum(nbv, 1.0), 0.0)
        ov[...] = lossv.astype(jnp.float32)
        pltpu.sync_copy(ov, out_hbm)


def kernel(input, target, label_weight):
    n = input.shape[0]
    mesh = plsc.VectorSubcoreMesh(
        core_axis_name="c", subcore_axis_name="s", num_cores=_NC, num_subcores=_NS
    )
    parts = pl.kernel(
        functools.partial(_hist_body, n),
        out_type=jax.ShapeDtypeStruct((_NW, _ROWS * _L), jnp.float32),
        mesh=mesh,
        scratch_types=[
            pltpu.VMEM((2, _CHUNK), jnp.float32),
            pltpu.VMEM((2, _CHUNK), jnp.int32),
            pltpu.VMEM((2, _CHUNK), jnp.float32),
            pltpu.VMEM((_ROWS * _L,), jnp.float32),
            pltpu.SemaphoreType.DMA,
            pltpu.SemaphoreType.DMA,
        ],
        compiler_params=pltpu.CompilerParams(needs_layout_passes=False),
    )(input, target, label_weight)
    out = pl.kernel(
        _combine_body,
        out_type=jax.ShapeDtypeStruct((_L,), jnp.float32),
        mesh=mesh,
        scratch_types=[
            pltpu.VMEM((_NW, _ROWS * _L), jnp.float32),
            pltpu.VMEM((_L,), jnp.float32),
            pltpu.SemaphoreType.DMA,
        ],
        compiler_params=pltpu.CompilerParams(needs_layout_passes=False),
    )(parts)
    return out[0]


# speedup vs baseline: 3.0907x; 3.0907x over previous
import functools

import jax
import jax.numpy as jnp
from jax import lax
from jax.experimental import pallas as pl
from jax.experimental.pallas import tpu as pltpu
from jax.experimental.pallas import tpu_sc as plsc

_NC = 2
_NS = 16
_NW = _NC * _NS
_L = 16
_BINS = 10
_ROWS = 2 * _BINS
_CHUNK = 8192
_UNROLL = 4
_TC_R = 512
_TC_C = 512
_TC_PAD = 4
_TC_FRAC_NUM, _TC_FRAC_DEN = 1, 2

_LOG1P = (
    2.2117031200252768e-05, 0.9990104466294587, -0.4891568472023044,
    0.28330432451740856, -0.13011941539126315, 0.03010262501167511,
)


def _log1p_poly(u):
    acc = jnp.float32(_LOG1P[5])
    for k in range(4, -1, -1):
        acc = acc * u + jnp.float32(_LOG1P[k])
    return acc


def _hist_body(n, m_off, x_hbm, t_hbm, w_hbm, parts_hbm, xb, tb, wb, hist,
               s0, s1):
    wid = lax.axis_index("s") * _NC + lax.axis_index("c")
    per_w = (n - m_off) // _NW
    nchunks = per_w // _CHUNK
    base = m_off + wid * per_w
    sems = (s0, s1)
    lane = lax.iota(jnp.int32, _L)

    for r in range(_UNROLL * _ROWS):
        hist[pl.ds(r * _L, _L)] = jnp.zeros((_L,), jnp.float32)

    def start(c, buf):
        off = base + c * _CHUNK
        sem = sems[buf]
        pltpu.make_async_copy(x_hbm.at[pl.ds(off, _CHUNK)], xb.at[buf], sem).start()
        pltpu.make_async_copy(t_hbm.at[pl.ds(off, _CHUNK)], tb.at[buf], sem).start()
        pltpu.make_async_copy(w_hbm.at[pl.ds(off, _CHUNK)], wb.at[buf], sem).start()

    def wait(buf):
        sem = sems[buf]
        pltpu.make_async_copy(x_hbm.at[pl.ds(0, _CHUNK)], xb.at[buf], sem).wait()
        pltpu.make_async_copy(t_hbm.at[pl.ds(0, _CHUNK)], tb.at[buf], sem).wait()
        pltpu.make_async_copy(w_hbm.at[pl.ds(0, _CHUNK)], wb.at[buf], sem).wait()

    ones = jnp.ones((_L,), jnp.float32)
    nu = _UNROLL

    def process(buf):
        def vbody(v, carry):
            s = v * (nu * _L)
            x = [xb[buf, pl.ds(s + u * _L, _L)] for u in range(nu)]
            tf = [tb[buf, pl.ds(s + u * _L, _L)].astype(jnp.float32)
                  for u in range(nu)]
            lw = [wb[buf, pl.ds(s + u * _L, _L)] for u in range(nu)]
            ax = [jnp.abs(xi) for xi in x]
            e = [jnp.exp(-a) for a in ax]
            r = [1.0 / (1.0 + ei) for ei in e]
            p = [jnp.where(x[u] >= 0, r[u], e[u] * r[u]) for u in range(nu)]
            g = [jnp.abs(p[u] - tf[u]) for u in range(nu)]
            bi = [jnp.minimum((gi * 10.0).astype(jnp.int32), _BINS - 1)
                  for gi in g]
            sp = [_log1p_poly(ei) for ei in e]
            pe = [jnp.maximum(x[u], 0.0) - x[u] * tf[u] + sp[u]
                  for u in range(nu)]
            valid = [lwi > 0 for lwi in lw]
            flat = [b * _L + lane for b in bi]
            for u in range(nu):
                base_u = u * _ROWS * _L
                plsc.addupdate_scatter(
                    hist, [flat[u] + base_u], ones, mask=valid[u])
                plsc.addupdate_scatter(
                    hist, [flat[u] + (base_u + _BINS * _L)], pe[u],
                    mask=valid[u])
            return carry

        lax.fori_loop(0, _CHUNK // (nu * _L), vbody, 0)

    start(0, 0)

    def cbody(i, carry):
        c0 = 2 * i
        wait(0)
        start(c0 + 1, 1)
        process(0)
        wait(1)

        @pl.when(c0 + 2 < nchunks)
        def _():
            start(c0 + 2, 0)

        process(1)
        return carry

    lax.fori_loop(0, nchunks // 2, cbody, 0)
    pltpu.sync_copy(hist, parts_hbm.at[wid])


def _tc_body(nsteps, x_ref, t_ref, w_ref, out_ref):
    i = pl.program_id(0)
    x = x_ref[...]
    tf = t_ref[...].astype(jnp.float32)
    lw = w_ref[...]
    ax = jnp.abs(x)
    e = jnp.exp(-ax)
    r = 1.0 / (1.0 + e)
    p = jnp.where(x >= 0, r, e * r)
    g = jnp.abs(p - tf)
    pe = jnp.maximum(x, 0.0) - x * tf + jnp.log(1.0 + e)
    validf = (lw > 0).astype(jnp.float32)
    pem = pe * validf
    cnts, sums = [], []
    for j in range(_BINS):
        if j == 0:
            indc, inds = validf, pem
        else:
            m = g >= jnp.float32(j / 10.0)
            indc = jnp.where(m, validf, 0.0)
            inds = jnp.where(m, pem, 0.0)
        cnts.append(jnp.sum(indc, axis=0, keepdims=True))
        sums.append(jnp.sum(inds, axis=0, keepdims=True))
    upd = jnp.concatenate(
        cnts + sums + [jnp.zeros((_TC_PAD, _TC_C), jnp.float32)], axis=0
    )

    @pl.when(i == 0)
    def _():
        out_ref[...] = jnp.zeros_like(out_ref)

    out_ref[...] += upd


def _combine_body(parts_hbm, tc_hbm, out_hbm, pv, tcv, ov, sem):
    wid = lax.axis_index("s") * _NC + lax.axis_index("c")

    @pl.when(wid == 0)
    def _():
        pltpu.make_async_copy(parts_hbm, pv, sem).start()
        pltpu.make_async_copy(tc_hbm, tcv, sem).start()
        pltpu.make_async_copy(parts_hbm, pv, sem).wait()
        pltpu.make_async_copy(tc_hbm, tcv, sem).wait()

        def wbody(w, accs):
            new = list(accs)
            for k in range(_UNROLL):
                for r in range(_ROWS):
                    new[r] = new[r] + pv[w, pl.ds((k * _ROWS + r) * _L, _L)]
            return tuple(new)

        zero = jnp.zeros((_L,), jnp.float32)
        accs = lax.fori_loop(0, _NW, wbody, tuple(zero for _ in range(_ROWS)))

        def tbody(k, taccs):
            return tuple(
                taccs[r] + tcv[r, pl.ds(k * _L, _L)] for r in range(_ROWS)
            )

        taccs = lax.fori_loop(
            0, _TC_C // _L, tbody, tuple(zero for _ in range(_ROWS))
        )
        c = [jnp.sum(taccs[r]) for r in range(_ROWS)]

        nbv = jnp.zeros((_L,), jnp.float32)
        contribv = jnp.zeros((_L,), jnp.float32)
        for b in range(_BINS):
            tc_cnt = c[b] - c[b + 1] if b < _BINS - 1 else c[_BINS - 1]
            tc_bs = (c[_BINS + b] - c[_BINS + b + 1]
                     if b < _BINS - 1 else c[2 * _BINS - 1])
            cntv = jnp.broadcast_to(jnp.sum(accs[b]) + tc_cnt, (_L,))
            bsv = jnp.broadcast_to(jnp.sum(accs[b + _BINS]) + tc_bs, (_L,))
            has = cntv > 0
            nbv = nbv + jnp.where(has, 1.0, 0.0).astype(jnp.float32)
            contribv = contribv + jnp.where(
                has, bsv / jnp.maximum(cntv, 1.0), 0.0
            ).astype(jnp.float32)
        lossv = jnp.where(nbv > 0, contribv / jnp.maximum(nbv, 1.0), 0.0)
        ov[...] = lossv.astype(jnp.float32)
        pltpu.sync_copy(ov, out_hbm)


def kernel(input, target, label_weight):
    n = input.shape[0]
    blk = _TC_R * _TC_C
    m = (n * _TC_FRAC_NUM // _TC_FRAC_DEN) // blk * blk
    nsteps = m // blk
    mesh = plsc.VectorSubcoreMesh(
        core_axis_name="c", subcore_axis_name="s", num_cores=_NC, num_subcores=_NS
    )
    parts = pl.kernel(
        functools.partial(_hist_body, n, m),
        out_type=jax.ShapeDtypeStruct((_NW, _UNROLL * _ROWS * _L), jnp.float32),
        mesh=mesh,
        scratch_types=[
            pltpu.VMEM((2, _CHUNK), jnp.float32),
            pltpu.VMEM((2, _CHUNK), jnp.int32),
            pltpu.VMEM((2, _CHUNK), jnp.float32),
            pltpu.VMEM((_UNROLL * _ROWS * _L,), jnp.float32),
            pltpu.SemaphoreType.DMA,
            pltpu.SemaphoreType.DMA,
        ],
        compiler_params=pltpu.CompilerParams(needs_layout_passes=False),
    )(input, target, label_weight)
    tc_cums = pl.pallas_call(
        functools.partial(_tc_body, nsteps),
        grid=(nsteps,),
        in_specs=[
            pl.BlockSpec((_TC_R, _TC_C), lambda i: (i, 0)),
            pl.BlockSpec((_TC_R, _TC_C), lambda i: (i, 0)),
            pl.BlockSpec((_TC_R, _TC_C), lambda i: (i, 0)),
        ],
        out_specs=pl.BlockSpec((_ROWS + _TC_PAD, _TC_C), lambda i: (0, 0)),
        out_shape=jax.ShapeDtypeStruct((_ROWS + _TC_PAD, _TC_C), jnp.float32),
    )(
        input.reshape(-1, _TC_C),
        target.reshape(-1, _TC_C),
        label_weight.reshape(-1, _TC_C),
    )
    out = pl.kernel(
        _combine_body,
        out_type=jax.ShapeDtypeStruct((_L,), jnp.float32),
        mesh=mesh,
        scratch_types=[
            pltpu.VMEM((_NW, _UNROLL * _ROWS * _L), jnp.float32),
            pltpu.VMEM((_ROWS + _TC_PAD, _TC_C), jnp.float32),
            pltpu.VMEM((_L,), jnp.float32),
            pltpu.SemaphoreType.DMA,
        ],
        compiler_params=pltpu.CompilerParams(needs_layout_passes=False),
    )(parts, tc_cums)
    return out[0]
